# baseline (speedup 1.0000x reference)
import jax
import jax.numpy as jnp
from jax import lax
from jax.experimental import pallas as pl
from jax.experimental.pallas import tpu as pltpu
from jax.experimental.pallas import tpu_sc as plsc

N = 10000
E = 320000
D = 128
C = 16

NC = 2
NS = 16
NW = NC * NS
L = 16

EPW = E // NW
RLEN = 625
ROWS = EPW // RLEN
NPW = N // NS

_MESH = plsc.VectorSubcoreMesh(
    core_axis_name="c", subcore_axis_name="s", num_cores=NC, num_subcores=NS
)
_SC_PARAMS = pltpu.CompilerParams(
    needs_layout_passes=False, use_tc_tiling_on_sc=False
)


def _deg_body(src, dst, dego_out, degi_out, sidx_v, didx_v, dego_v, degi_v):
    c = lax.axis_index("c")
    s = lax.axis_index("s")
    wid = c * NS + s
    pltpu.sync_copy(src.at[wid], sidx_v)
    pltpu.sync_copy(dst.at[wid], didx_v)

    zero16 = jnp.zeros((L,), jnp.float32)
    ones16 = jnp.ones((L,), jnp.float32)

    def zb(i, carry):
        dego_v[pl.ds(i * L, L)] = zero16
        degi_v[pl.ds(i * L, L)] = zero16
        return carry

    lax.fori_loop(0, N // L, zb, 0)

    def eb(i, carry):
        sv = sidx_v[pl.ds(i * L, L)]
        plsc.addupdate_scatter(dego_v, [sv], ones16)
        dv = didx_v[pl.ds(i * L, L)]
        plsc.addupdate_scatter(degi_v, [dv], ones16)
        return carry

    lax.fori_loop(0, EPW // L, eb, 0)

    pltpu.sync_copy(dego_v, dego_out.at[wid])
    pltpu.sync_copy(degi_v, degi_out.at[wid])


_deg_call = pl.kernel(
    _deg_body,
    out_type=(
        jax.ShapeDtypeStruct((NW, N), jnp.float32),
        jax.ShapeDtypeStruct((NW, N), jnp.float32),
    ),
    mesh=_MESH,
    scratch_types=[
        pltpu.VMEM((EPW,), jnp.int32),
        pltpu.VMEM((EPW,), jnp.int32),
        pltpu.VMEM((N,), jnp.float32),
        pltpu.VMEM((N,), jnp.float32),
    ],
    compiler_params=_SC_PARAMS,
)


BN = 1000


def _mm_body(x_ref, w_ref, h0_ref):
    h0_ref[...] = jnp.dot(
        x_ref[...], w_ref[...], preferred_element_type=jnp.float32
    )


_mm_call = pl.pallas_call(
    _mm_body,
    grid=(N // BN,),
    in_specs=[
        pl.BlockSpec((BN, D), lambda i: (i, 0)),
        pl.BlockSpec((D, C), lambda i: (0, 0)),
    ],
    out_specs=pl.BlockSpec((BN, C), lambda i: (i, 0)),
    out_shape=jax.ShapeDtypeStruct((N, C), jnp.float32),
)


def _scale_body(h0_ref, dgo_ref, dgi_ref, h_ref, nde_ref):
    ns = lax.rsqrt(jnp.maximum(jnp.sum(dgo_ref[...], axis=1), 1.0))
    h_ref[...] = h0_ref[...] * ns[:, None]
    nd = lax.rsqrt(jnp.maximum(jnp.sum(dgi_ref[...], axis=1), 1.0))
    nde_ref[...] = jnp.broadcast_to(nd[:, None], (BN, C))


_scale_call = pl.pallas_call(
    _scale_body,
    grid=(N // BN,),
    in_specs=[
        pl.BlockSpec((BN, C), lambda i: (i, 0)),
        pl.BlockSpec((BN, NW), lambda i: (i, 0)),
        pl.BlockSpec((BN, NW), lambda i: (i, 0)),
    ],
    out_specs=[
        pl.BlockSpec((BN, C), lambda i: (i, 0)),
        pl.BlockSpec((BN, C), lambda i: (i, 0)),
    ],
    out_shape=[
        jax.ShapeDtypeStruct((N, C), jnp.float32),
        jax.ShapeDtypeStruct((N, C), jnp.float32),
    ],
)


def _agg_body(
    h, src_r, dst_r, aggp, sidx_v, didx_v, msg_v, msg2_v, buf_v, agg_s, sem, sem2
):
    c = lax.axis_index("c")
    s = lax.axis_index("s")
    wid = c * NS + s
    pltpu.sync_copy(src_r.at[wid], sidx_v)
    pltpu.sync_copy(dst_r.at[wid], didx_v)

    def zb(i, carry):
        buf_v[i, :] = jnp.zeros((C,), jnp.float32)
        return carry

    lax.fori_loop(0, NPW, zb, 0)
    pltpu.sync_copy(buf_v, agg_s.at[pl.ds(s * NPW, NPW)])
    plsc.subcore_barrier()

    pltpu.async_copy(h.at[sidx_v.at[0]], msg_v, sem)

    def eb(jj, carry):
        j0 = 2 * jj
        j1 = j0 + 1
        pltpu.async_copy(h.at[sidx_v.at[j1]], msg2_v, sem2)
        pltpu.make_async_copy(h.at[sidx_v.at[j0]], msg_v, sem).wait()
        pltpu.sync_copy(msg_v, agg_s.at[didx_v.at[j0]], add=True)

        @pl.when(jj + 1 < ROWS // 2)
        def _():
            pltpu.async_copy(h.at[sidx_v.at[j0 + 2]], msg_v, sem)

        pltpu.make_async_copy(h.at[sidx_v.at[j1]], msg2_v, sem2).wait()
        pltpu.sync_copy(msg2_v, agg_s.at[didx_v.at[j1]], add=True)
        return carry

    lax.fori_loop(0, ROWS // 2, eb, 0)
    plsc.subcore_barrier()

    pltpu.sync_copy(agg_s.at[pl.ds(s * NPW, NPW)], buf_v)
    pltpu.sync_copy(buf_v, aggp.at[c, s])


_agg_call = pl.kernel(
    _agg_body,
    out_type=jax.ShapeDtypeStruct((NC, NS, NPW, C), jnp.float32),
    mesh=_MESH,
    scratch_types=[
        pltpu.VMEM((ROWS, RLEN), jnp.int32),
        pltpu.VMEM((ROWS, RLEN), jnp.int32),
        pltpu.VMEM((RLEN, C), jnp.float32),
        pltpu.VMEM((RLEN, C), jnp.float32),
        pltpu.VMEM((NPW, C), jnp.float32),
        pltpu.VMEM_SHARED((N, C), jnp.float32),
        pltpu.SemaphoreType.DMA,
        pltpu.SemaphoreType.DMA,
    ],
    compiler_params=_SC_PARAMS,
)


R0 = 312
R1 = NPW - R0


def _fin_body(aggp, nde, bias, out, a0_v, a1_v, nd_v, b_v, o_v):
    c = lax.axis_index("c")
    s = lax.axis_index("s")
    pltpu.sync_copy(bias, b_v)
    bvec = b_v[...]

    def run(off, nr):
        r0 = s * NPW + off
        pltpu.sync_copy(aggp.at[0, s, pl.ds(off, nr)], a0_v.at[pl.ds(0, nr)])
        pltpu.sync_copy(aggp.at[1, s, pl.ds(off, nr)], a1_v.at[pl.ds(0, nr)])
        pltpu.sync_copy(nde.at[pl.ds(r0, nr)], nd_v.at[pl.ds(0, nr)])

        def rb(r, carry):
            o_v[r, :] = (a0_v[r, :] + a1_v[r, :]) * nd_v[r, :] + bvec
            return carry

        lax.fori_loop(0, nr, rb, 0)
        pltpu.sync_copy(o_v.at[pl.ds(0, nr)], out.at[pl.ds(r0, nr)])

    @pl.when(c == 0)
    def _():
        run(0, R0)

    @pl.when(c == 1)
    def _():
        run(R0, R1)


_fin_call = pl.kernel(
    _fin_body,
    out_type=jax.ShapeDtypeStruct((N, C), jnp.float32),
    mesh=_MESH,
    scratch_types=[
        pltpu.VMEM((R1, C), jnp.float32),
        pltpu.VMEM((R1, C), jnp.float32),
        pltpu.VMEM((R1, C), jnp.float32),
        pltpu.VMEM((C,), jnp.float32),
        pltpu.VMEM((R1, C), jnp.float32),
    ],
    compiler_params=_SC_PARAMS,
)


def kernel(in_feat, edge_index, W, b):
    src2 = edge_index[0].reshape(NW, EPW)
    dst2 = edge_index[1].reshape(NW, EPW)
    dego_p, degi_p = _deg_call(src2, dst2)
    h0 = _mm_call(in_feat, W)
    h, nde = _scale_call(h0, dego_p.T, degi_p.T)
    aggp = _agg_call(
        h, src2.reshape(NW, ROWS, RLEN), dst2.reshape(NW, ROWS, RLEN)
    )
    return _fin_call(aggp, nde, b)


# speedup vs baseline: 1.0393x; 1.0393x over previous
import jax
import jax.numpy as jnp
from jax import lax
from jax.experimental import pallas as pl
from jax.experimental.pallas import tpu as pltpu
from jax.experimental.pallas import tpu_sc as plsc

N = 10000
E = 320000
D = 128
C = 16

NC = 2
NS = 16
NW = NC * NS
L = 16

EPW = E // NW
RLEN = 625
ROWS = EPW // RLEN
NPW = N // NS

_MESH = plsc.VectorSubcoreMesh(
    core_axis_name="c", subcore_axis_name="s", num_cores=NC, num_subcores=NS
)
_SC_PARAMS = pltpu.CompilerParams(
    needs_layout_passes=False, use_tc_tiling_on_sc=False
)


CW = 640
CWL = N - 15 * CW


def _deg_body(
    src, dst, dego_out, degi_out,
    sidx_v, didx_v, dego_v, degi_v, rbuf_v, ov_v, dego_sh, degi_sh,
):
    c = lax.axis_index("c")
    s = lax.axis_index("s")
    wid = c * NS + s
    pltpu.sync_copy(src.at[wid], sidx_v)
    pltpu.sync_copy(dst.at[wid], didx_v)

    zero16 = jnp.zeros((L,), jnp.float32)
    ones16 = jnp.ones((L,), jnp.float32)

    def zb(i, carry):
        dego_v[pl.ds(i * L, L)] = zero16
        degi_v[pl.ds(i * L, L)] = zero16
        return carry

    lax.fori_loop(0, N // L, zb, 0)

    def eb(i, carry):
        sv = sidx_v[pl.ds(i * L, L)]
        plsc.addupdate_scatter(dego_v, [sv], ones16)
        dv = didx_v[pl.ds(i * L, L)]
        plsc.addupdate_scatter(degi_v, [dv], ones16)
        return carry

    lax.fori_loop(0, EPW // L, eb, 0)

    pltpu.sync_copy(dego_v, dego_sh.at[s])
    pltpu.sync_copy(degi_v, degi_sh.at[s])
    plsc.subcore_barrier()

    def reduce_block(sh, out, off, width):
        pltpu.sync_copy(sh.at[:, pl.ds(off, width)], rbuf_v.at[:, pl.ds(0, width)])

        def gb(g, carry):
            acc = rbuf_v[0, pl.ds(g * L, L)]
            for w in range(1, NS):
                acc = acc + rbuf_v[w, pl.ds(g * L, L)]
            ov_v[pl.ds(g * L, L)] = acc
            return carry

        lax.fori_loop(0, width // L, gb, 0)
        pltpu.sync_copy(ov_v.at[pl.ds(0, width)], out.at[c, pl.ds(off, width)])

    @pl.when(s < 15)
    def _():
        reduce_block(dego_sh, dego_out, s * CW, CW)
        reduce_block(degi_sh, degi_out, s * CW, CW)

    @pl.when(s == 15)
    def _():
        reduce_block(dego_sh, dego_out, 15 * CW, CWL)
        reduce_block(degi_sh, degi_out, 15 * CW, CWL)


_deg_call = pl.kernel(
    _deg_body,
    out_type=(
        jax.ShapeDtypeStruct((NC, N), jnp.float32),
        jax.ShapeDtypeStruct((NC, N), jnp.float32),
    ),
    mesh=_MESH,
    scratch_types=[
        pltpu.VMEM((EPW,), jnp.int32),
        pltpu.VMEM((EPW,), jnp.int32),
        pltpu.VMEM((N,), jnp.float32),
        pltpu.VMEM((N,), jnp.float32),
        pltpu.VMEM((NS, CW), jnp.float32),
        pltpu.VMEM((CW,), jnp.float32),
        pltpu.VMEM_SHARED((NS, N), jnp.float32),
        pltpu.VMEM_SHARED((NS, N), jnp.float32),
    ],
    compiler_params=_SC_PARAMS,
)


BN = 1000


def _mid_body(x_ref, w_ref, dgo_ref, dgi_ref, h_ref, nd_ref):
    h0 = jnp.dot(x_ref[...], w_ref[...], preferred_element_type=jnp.float32)
    ns = lax.rsqrt(jnp.maximum(jnp.sum(dgo_ref[...], axis=1), 1.0))
    h_ref[...] = h0 * ns[:, None]
    nd = lax.rsqrt(jnp.maximum(jnp.sum(dgi_ref[...], axis=1), 1.0))
    nd_ref[...] = jnp.broadcast_to(nd[:, None], (BN, C))


_mid_call = pl.pallas_call(
    _mid_body,
    grid=(N // BN,),
    in_specs=[
        pl.BlockSpec((BN, D), lambda i: (i, 0)),
        pl.BlockSpec((D, C), lambda i: (0, 0)),
        pl.BlockSpec((BN, NC), lambda i: (i, 0)),
        pl.BlockSpec((BN, NC), lambda i: (i, 0)),
    ],
    out_specs=[
        pl.BlockSpec((BN, C), lambda i: (i, 0)),
        pl.BlockSpec((BN, C), lambda i: (i, 0)),
    ],
    out_shape=[
        jax.ShapeDtypeStruct((N, C), jnp.float32),
        jax.ShapeDtypeStruct((N, C), jnp.float32),
    ],
)


def _agg_body(
    h, src_r, dst_r, aggp, sidx_v, didx_v, msg_v, msg2_v, buf_v, agg_s, sem, sem2
):
    c = lax.axis_index("c")
    s = lax.axis_index("s")
    wid = c * NS + s
    pltpu.sync_copy(src_r.at[wid], sidx_v)
    pltpu.sync_copy(dst_r.at[wid], didx_v)

    def zb(i, carry):
        buf_v[i, :] = jnp.zeros((C,), jnp.float32)
        return carry

    lax.fori_loop(0, NPW, zb, 0)
    pltpu.sync_copy(buf_v, agg_s.at[pl.ds(s * NPW, NPW)])
    plsc.subcore_barrier()

    pltpu.async_copy(h.at[sidx_v.at[0]], msg_v, sem)

    def eb(jj, carry):
        j0 = 2 * jj
        j1 = j0 + 1
        pltpu.async_copy(h.at[sidx_v.at[j1]], msg2_v, sem2)
        pltpu.make_async_copy(h.at[sidx_v.at[j0]], msg_v, sem).wait()
        pltpu.sync_copy(msg_v, agg_s.at[didx_v.at[j0]], add=True)

        @pl.when(jj + 1 < ROWS // 2)
        def _():
            pltpu.async_copy(h.at[sidx_v.at[j0 + 2]], msg_v, sem)

        pltpu.make_async_copy(h.at[sidx_v.at[j1]], msg2_v, sem2).wait()
        pltpu.sync_copy(msg2_v, agg_s.at[didx_v.at[j1]], add=True)
        return carry

    lax.fori_loop(0, ROWS // 2, eb, 0)
    plsc.subcore_barrier()

    pltpu.sync_copy(agg_s.at[pl.ds(s * NPW, NPW)], buf_v)
    pltpu.sync_copy(buf_v, aggp.at[c, s])


_agg_call = pl.kernel(
    _agg_body,
    out_type=jax.ShapeDtypeStruct((NC, NS, NPW, C), jnp.float32),
    mesh=_MESH,
    scratch_types=[
        pltpu.VMEM((ROWS, RLEN), jnp.int32),
        pltpu.VMEM((ROWS, RLEN), jnp.int32),
        pltpu.VMEM((RLEN, C), jnp.float32),
        pltpu.VMEM((RLEN, C), jnp.float32),
        pltpu.VMEM((NPW, C), jnp.float32),
        pltpu.VMEM_SHARED((N, C), jnp.float32),
        pltpu.SemaphoreType.DMA,
        pltpu.SemaphoreType.DMA,
    ],
    compiler_params=_SC_PARAMS,
)


R0 = 312
R1 = NPW - R0


def _fin_body(aggp, nd2, bias, out, a0_v, a1_v, nd_v, b_v, o_v):
    c = lax.axis_index("c")
    s = lax.axis_index("s")
    pltpu.sync_copy(bias, b_v)
    bvec = b_v[...]

    def run(off, nr):
        r0 = s * NPW + off
        pltpu.sync_copy(aggp.at[0, s, pl.ds(off, nr)], a0_v.at[pl.ds(0, nr)])
        pltpu.sync_copy(aggp.at[1, s, pl.ds(off, nr)], a1_v.at[pl.ds(0, nr)])
        pltpu.sync_copy(nd2.at[pl.ds(s * NPW + off, nr)], nd_v.at[pl.ds(0, nr)])

        def rb(r, carry):
            o_v[r, :] = (a0_v[r, :] + a1_v[r, :]) * nd_v[r, :] + bvec
            return carry

        lax.fori_loop(0, nr, rb, 0)
        pltpu.sync_copy(o_v.at[pl.ds(0, nr)], out.at[pl.ds(r0, nr)])

    @pl.when(c == 0)
    def _():
        run(0, R0)

    @pl.when(c == 1)
    def _():
        run(R0, R1)


_fin_call = pl.kernel(
    _fin_body,
    out_type=jax.ShapeDtypeStruct((N, C), jnp.float32),
    mesh=_MESH,
    scratch_types=[
        pltpu.VMEM((R1, C), jnp.float32),
        pltpu.VMEM((R1, C), jnp.float32),
        pltpu.VMEM((R1, C), jnp.float32),
        pltpu.VMEM((C,), jnp.float32),
        pltpu.VMEM((R1, C), jnp.float32),
    ],
    compiler_params=_SC_PARAMS,
)


def kernel(in_feat, edge_index, W, b):
    src2 = edge_index[0].reshape(NW, EPW)
    dst2 = edge_index[1].reshape(NW, EPW)
    dego_p, degi_p = _deg_call(src2, dst2)
    h, nde = _mid_call(in_feat, W, dego_p.T, degi_p.T)
    aggp = _agg_call(
        h, src2.reshape(NW, ROWS, RLEN), dst2.reshape(NW, ROWS, RLEN)
    )
    return _fin_call(aggp, nde, b)


# speedup vs baseline: 1.1760x; 1.1315x over previous
import jax
import jax.numpy as jnp
from jax import lax
from jax.experimental import pallas as pl
from jax.experimental.pallas import tpu as pltpu
from jax.experimental.pallas import tpu_sc as plsc

N = 10000
E = 320000
D = 128
C = 16

NC = 2
NS = 16
NW = NC * NS
L = 16

EPW = E // NW
RLEN = 1250
ROWS = EPW // RLEN
NPW = N // NS

_MESH = plsc.VectorSubcoreMesh(
    core_axis_name="c", subcore_axis_name="s", num_cores=NC, num_subcores=NS
)
_SC_PARAMS = pltpu.CompilerParams(
    needs_layout_passes=False, use_tc_tiling_on_sc=False
)


CW = 640
CWL = N - 15 * CW


def _deg_body(
    src, dst, dego_out, degi_out,
    sidx_v, didx_v, dego_v, degi_v, rbuf_v, ov_v, dego_sh, degi_sh,
):
    c = lax.axis_index("c")
    s = lax.axis_index("s")
    wid = c * NS + s
    pltpu.sync_copy(src.at[wid], sidx_v)
    pltpu.sync_copy(dst.at[wid], didx_v)

    zero16 = jnp.zeros((L,), jnp.float32)
    ones16 = jnp.ones((L,), jnp.float32)

    def zb(i, carry):
        base = i * (5 * L)
        for u in range(5):
            dego_v[pl.ds(base + u * L, L)] = zero16
            degi_v[pl.ds(base + u * L, L)] = zero16
        return carry

    lax.fori_loop(0, N // (5 * L), zb, 0)

    def eb(i, carry):
        base = i * (5 * L)
        for u in range(5):
            sv = sidx_v[pl.ds(base + u * L, L)]
            plsc.addupdate_scatter(dego_v, [sv], ones16)
            dv = didx_v[pl.ds(base + u * L, L)]
            plsc.addupdate_scatter(degi_v, [dv], ones16)
        return carry

    lax.fori_loop(0, EPW // (5 * L), eb, 0)

    pltpu.sync_copy(dego_v, dego_sh.at[s])
    pltpu.sync_copy(degi_v, degi_sh.at[s])
    plsc.subcore_barrier()

    def reduce_block(sh, out, off, width):
        pltpu.sync_copy(sh.at[:, pl.ds(off, width)], rbuf_v.at[:, pl.ds(0, width)])

        def gb(g, carry):
            acc = rbuf_v[0, pl.ds(g * L, L)]
            for w in range(1, NS):
                acc = acc + rbuf_v[w, pl.ds(g * L, L)]
            ov_v[pl.ds(g * L, L)] = acc
            return carry

        lax.fori_loop(0, width // L, gb, 0)
        pltpu.sync_copy(ov_v.at[pl.ds(0, width)], out.at[c, pl.ds(off, width)])

    @pl.when(s < 15)
    def _():
        reduce_block(dego_sh, dego_out, s * CW, CW)
        reduce_block(degi_sh, degi_out, s * CW, CW)

    @pl.when(s == 15)
    def _():
        reduce_block(dego_sh, dego_out, 15 * CW, CWL)
        reduce_block(degi_sh, degi_out, 15 * CW, CWL)


_deg_call = pl.kernel(
    _deg_body,
    out_type=(
        jax.ShapeDtypeStruct((NC, N), jnp.float32),
        jax.ShapeDtypeStruct((NC, N), jnp.float32),
    ),
    mesh=_MESH,
    scratch_types=[
        pltpu.VMEM((EPW,), jnp.int32),
        pltpu.VMEM((EPW,), jnp.int32),
        pltpu.VMEM((N,), jnp.float32),
        pltpu.VMEM((N,), jnp.float32),
        pltpu.VMEM((NS, CW), jnp.float32),
        pltpu.VMEM((CW,), jnp.float32),
        pltpu.VMEM_SHARED((NS, N), jnp.float32),
        pltpu.VMEM_SHARED((NS, N), jnp.float32),
    ],
    compiler_params=_SC_PARAMS,
)


BN = 1000


def _mid_body(x_ref, w_ref, dgo_ref, dgi_ref, h_ref, nd_ref):
    h0 = jnp.dot(x_ref[...], w_ref[...], preferred_element_type=jnp.float32)
    ns = lax.rsqrt(jnp.maximum(jnp.sum(dgo_ref[...], axis=0), 1.0))
    h_ref[...] = h0 * ns[:, None]
    nd = lax.rsqrt(jnp.maximum(jnp.sum(dgi_ref[...], axis=0), 1.0))
    nd_ref[...] = jnp.broadcast_to(nd[:, None], (N, C))


_mid_call = pl.pallas_call(
    _mid_body,
    out_shape=[
        jax.ShapeDtypeStruct((N, C), jnp.float32),
        jax.ShapeDtypeStruct((N, C), jnp.float32),
    ],
)


def _agg_body(
    h, src_r, dst_r, aggp, sidx_v, didx_v, msg_v, msg2_v, buf_v, agg_s, sem, sem2
):
    c = lax.axis_index("c")
    s = lax.axis_index("s")
    wid = c * NS + s
    pltpu.sync_copy(src_r.at[wid], sidx_v)
    pltpu.sync_copy(dst_r.at[wid], didx_v)

    def zb(i, carry):
        buf_v[i, :] = jnp.zeros((C,), jnp.float32)
        return carry

    lax.fori_loop(0, NPW, zb, 0)
    pltpu.sync_copy(buf_v, agg_s.at[pl.ds(s * NPW, NPW)])
    plsc.subcore_barrier()

    pltpu.async_copy(h.at[sidx_v.at[0]], msg_v, sem)

    def eb(jj, carry):
        j0 = 2 * jj
        j1 = j0 + 1
        pltpu.async_copy(h.at[sidx_v.at[j1]], msg2_v, sem2)
        pltpu.make_async_copy(h.at[sidx_v.at[j0]], msg_v, sem).wait()
        pltpu.sync_copy(msg_v, agg_s.at[didx_v.at[j0]], add=True)

        @pl.when(jj + 1 < ROWS // 2)
        def _():
            pltpu.async_copy(h.at[sidx_v.at[j0 + 2]], msg_v, sem)

        pltpu.make_async_copy(h.at[sidx_v.at[j1]], msg2_v, sem2).wait()
        pltpu.sync_copy(msg2_v, agg_s.at[didx_v.at[j1]], add=True)
        return carry

    lax.fori_loop(0, ROWS // 2, eb, 0)
    plsc.subcore_barrier()

    pltpu.sync_copy(agg_s.at[pl.ds(s * NPW, NPW)], buf_v)
    pltpu.sync_copy(buf_v, aggp.at[c, s])


_agg_call = pl.kernel(
    _agg_body,
    out_type=jax.ShapeDtypeStruct((NC, NS, NPW, C), jnp.float32),
    mesh=_MESH,
    scratch_types=[
        pltpu.VMEM((ROWS, RLEN), jnp.int32),
        pltpu.VMEM((ROWS, RLEN), jnp.int32),
        pltpu.VMEM((RLEN, C), jnp.float32),
        pltpu.VMEM((RLEN, C), jnp.float32),
        pltpu.VMEM((NPW, C), jnp.float32),
        pltpu.VMEM_SHARED((N, C), jnp.float32),
        pltpu.SemaphoreType.DMA,
        pltpu.SemaphoreType.DMA,
    ],
    compiler_params=_SC_PARAMS,
)


R0 = 312
R1 = NPW - R0


def _fin_body(aggp, nd2, bias, out, a0_v, a1_v, nd_v, b_v, o_v):
    c = lax.axis_index("c")
    s = lax.axis_index("s")
    pltpu.sync_copy(bias, b_v)
    bvec = b_v[...]

    def run(off, nr):
        r0 = s * NPW + off
        pltpu.sync_copy(aggp.at[0, s, pl.ds(off, nr)], a0_v.at[pl.ds(0, nr)])
        pltpu.sync_copy(aggp.at[1, s, pl.ds(off, nr)], a1_v.at[pl.ds(0, nr)])
        pltpu.sync_copy(nd2.at[pl.ds(s * NPW + off, nr)], nd_v.at[pl.ds(0, nr)])

        def rb(r, carry):
            o_v[r, :] = (a0_v[r, :] + a1_v[r, :]) * nd_v[r, :] + bvec
            return carry

        lax.fori_loop(0, nr, rb, 0)
        pltpu.sync_copy(o_v.at[pl.ds(0, nr)], out.at[pl.ds(r0, nr)])

    @pl.when(c == 0)
    def _():
        run(0, R0)

    @pl.when(c == 1)
    def _():
        run(R0, R1)


_fin_call = pl.kernel(
    _fin_body,
    out_type=jax.ShapeDtypeStruct((N, C), jnp.float32),
    mesh=_MESH,
    scratch_types=[
        pltpu.VMEM((R1, C), jnp.float32),
        pltpu.VMEM((R1, C), jnp.float32),
        pltpu.VMEM((R1, C), jnp.float32),
        pltpu.VMEM((C,), jnp.float32),
        pltpu.VMEM((R1, C), jnp.float32),
    ],
    compiler_params=_SC_PARAMS,
)


def kernel(in_feat, edge_index, W, b):
    src2 = edge_index[0].reshape(NW, EPW)
    dst2 = edge_index[1].reshape(NW, EPW)
    dego_p, degi_p = _deg_call(src2, dst2)
    h, nde = _mid_call(in_feat, W, dego_p, degi_p)
    aggp = _agg_call(
        h, src2.reshape(NW, ROWS, RLEN), dst2.reshape(NW, ROWS, RLEN)
    )
    return _fin_call(aggp, nde, b)
